# baseline (speedup 1.0000x reference)
import functools

import jax
import jax.numpy as jnp
from jax import lax
from jax.experimental import pallas as pl
from jax.experimental.pallas import tpu as pltpu
from jax.experimental.pallas import tpu_sc as plsc

B = 16384
D = 64
L = 16


def _worker_id():
    return lax.axis_index("s") * 2 + lax.axis_index("c")


def _sc_ce_body(b_per_w, ce_t, cid, o_ce, idx_v, rows_v, sem):
    base = _worker_id() * b_per_w
    pltpu.sync_copy(cid.at[pl.ds(base, b_per_w)], idx_v)
    pltpu.async_copy(ce_t.at[idx_v], rows_v, sem).wait()
    pltpu.sync_copy(rows_v, o_ce.at[pl.ds(base, b_per_w)])


@functools.lru_cache(maxsize=None)
def _make_sc_ce():
    info = plsc.get_sparse_core_info()
    b_per_w = B // (info.num_cores * info.num_subcores)
    mesh = plsc.VectorSubcoreMesh(core_axis_name="c", subcore_axis_name="s")
    return pl.kernel(
        functools.partial(_sc_ce_body, b_per_w),
        mesh=mesh,
        compiler_params=pltpu.CompilerParams(use_tc_tiling_on_sc=False),
        out_type=jax.ShapeDtypeStruct((B, D), jnp.float32),
        scratch_types=[
            pltpu.VMEM((b_per_w,), jnp.int32),
            pltpu.VMEM((b_per_w, D), jnp.float32),
            pltpu.SemaphoreType.DMA,
        ],
    )


def _sc_rest_body(b_per_w, pe_t, ty_t, ca_t, pid, tid, gid,
                  o_pe, o_ty, o_ca, idx_v, rows_v, sem):
    base = _worker_id() * b_per_w
    for tab, idx, out in ((pe_t, pid, o_pe), (ty_t, tid, o_ty),
                          (ca_t, gid, o_ca)):
        pltpu.sync_copy(idx.at[pl.ds(base, b_per_w)], idx_v)

        def issue(j, _, tab=tab):
            vec = idx_v[pl.ds(j * L, L)]
            for k in range(L):
                pltpu.async_copy(tab.at[pl.ds(vec[k], 1)],
                                 rows_v.at[pl.ds(j * L + k, 1)], sem)
            return _

        lax.fori_loop(0, b_per_w // L, issue, 0, unroll=False)
        pltpu.make_async_copy(tab.at[pl.ds(0, b_per_w)], rows_v, sem).wait()
        pltpu.sync_copy(rows_v, out.at[pl.ds(base, b_per_w)])


@functools.lru_cache(maxsize=None)
def _make_sc_rest():
    info = plsc.get_sparse_core_info()
    b_per_w = B // (info.num_cores * info.num_subcores)
    mesh = plsc.VectorSubcoreMesh(core_axis_name="c", subcore_axis_name="s")
    return pl.kernel(
        functools.partial(_sc_rest_body, b_per_w),
        mesh=mesh,
        out_type=[jax.ShapeDtypeStruct((B, D), jnp.float32)] * 3,
        scratch_types=[
            pltpu.VMEM((b_per_w,), jnp.int32),
            pltpu.VMEM((b_per_w, D), jnp.float32),
            pltpu.SemaphoreType.DMA,
        ],
    )


def _mlp_body(ce, pe, ty, ca, pw, pp,
              pw_w, pw_b, pp_w, pp_b,
              fc1_w, fc1_b, fc2_w, fc2_b, out_w, out_b, o_ref):
    f32 = jnp.float32
    dn = (((1,), (1,)), ((), ()))

    def bdot(a, b):
        return lax.dot_general(a.astype(jnp.bfloat16), b.astype(jnp.bfloat16),
                               dn, preferred_element_type=f32)

    power_emb = pw[...] * pw_w[...] + pw_b[...]
    price_emb = pp[...] * pp_w[...] + pp_b[...]
    x = jnp.concatenate(
        [ce[...], pe[...], ty[...], power_emb, ca[...], price_emb], axis=-1)
    h = jnp.maximum(bdot(x, fc1_w[...]) + fc1_b[...], 0.0)
    h = jnp.maximum(bdot(h, fc2_w[...]) + fc2_b[...], 0.0)
    o = jnp.sum(h * out_w[...], axis=1)
    o_ref[...] = o + out_b[0]


def _mlp(ce, pe, ty, ca, pw2, pp2, pw_w, pw_b, pp_w, pp_b,
         fc1_w, fc1_b, fc2_w, fc2_b, out_w, out_b):
    blk = 2048
    grid = (B // blk,)

    def row_spec(d):
        return pl.BlockSpec((blk, d), lambda i: (i, 0))

    def full_spec(shape):
        nd = len(shape)
        return pl.BlockSpec(shape, (lambda i: (0,) * nd))

    in_specs = [
        row_spec(D), row_spec(D), row_spec(D), row_spec(D),
        row_spec(1), row_spec(1),
        full_spec(pw_w.shape), full_spec(pw_b.shape),
        full_spec(pp_w.shape), full_spec(pp_b.shape),
        full_spec(fc1_w.shape), full_spec(fc1_b.shape),
        full_spec(fc2_w.shape), full_spec(fc2_b.shape),
        full_spec(out_w.shape), full_spec(out_b.shape),
    ]
    return pl.pallas_call(
        _mlp_body,
        grid=grid,
        in_specs=in_specs,
        out_specs=pl.BlockSpec((blk,), lambda i: (i,)),
        out_shape=jax.ShapeDtypeStruct((B,), jnp.float32),
    )(ce, pe, ty, ca, pw2, pp2, pw_w, pw_b, pp_w, pp_b,
      fc1_w, fc1_b, fc2_w, fc2_b, out_w, out_b)


def kernel(customer_id, product_id, customer_type, purchasing_power,
           product_category, product_price,
           ce_table, pe_table, type_table, cat_table,
           pw_w, pw_b, pp_w, pp_b,
           fc1_w, fc1_b, fc2_w, fc2_b, out_w, out_b):
    ce = _make_sc_ce()(ce_table, customer_id)
    pe, ty, ca = _make_sc_rest()(
        pe_table, type_table, cat_table,
        product_id, customer_type, product_category)
    return _mlp(ce, pe, ty, ca,
                purchasing_power[:, None], product_price[:, None],
                pw_w.T, pw_b, pp_w.T, pp_b,
                fc1_w, fc1_b, fc2_w, fc2_b, out_w, out_b)


# speedup vs baseline: 1.5781x; 1.5781x over previous
import functools

import jax
import jax.numpy as jnp
from jax import lax
from jax.experimental import pallas as pl
from jax.experimental.pallas import tpu as pltpu
from jax.experimental.pallas import tpu_sc as plsc

B = 16384
D = 64
L = 16


def _worker_id():
    return lax.axis_index("s") * 2 + lax.axis_index("c")


def _sc_ce_body(b_per_w, ce_t, cid, o_ce, idx_v, rows_v, sem):
    base = _worker_id() * b_per_w
    pltpu.sync_copy(cid.at[pl.ds(base, b_per_w)], idx_v)

    def issue(j, _):
        vec = idx_v[pl.ds(j * L, L)]
        for k in range(L):
            pltpu.async_copy(ce_t.at[pl.ds(vec[k], 1)],
                             rows_v.at[pl.ds(j * L + k, 1)], sem)
        return _

    lax.fori_loop(0, b_per_w // L, issue, 0, unroll=False)
    pltpu.make_async_copy(ce_t.at[pl.ds(0, b_per_w)], rows_v, sem).wait()
    pltpu.sync_copy(rows_v, o_ce.at[pl.ds(base, b_per_w)])


@functools.lru_cache(maxsize=None)
def _make_sc_ce():
    info = plsc.get_sparse_core_info()
    b_per_w = B // (info.num_cores * info.num_subcores)
    mesh = plsc.VectorSubcoreMesh(core_axis_name="c", subcore_axis_name="s")
    return pl.kernel(
        functools.partial(_sc_ce_body, b_per_w),
        mesh=mesh,
        out_type=jax.ShapeDtypeStruct((B, D), jnp.float32),
        scratch_types=[
            pltpu.VMEM((b_per_w,), jnp.int32),
            pltpu.VMEM((b_per_w, D), jnp.float32),
            pltpu.SemaphoreType.DMA,
        ],
    )


def _sc_rest_body(b_per_w, pe_t, ty_t, ca_t, pid, tid, gid,
                  o_pe, o_ty, o_ca, idx_v, rows_v, sem):
    base = _worker_id() * b_per_w
    for tab, idx, out in ((pe_t, pid, o_pe), (ty_t, tid, o_ty),
                          (ca_t, gid, o_ca)):
        pltpu.sync_copy(idx.at[pl.ds(base, b_per_w)], idx_v)

        def issue(j, _, tab=tab):
            vec = idx_v[pl.ds(j * L, L)]
            for k in range(L):
                pltpu.async_copy(tab.at[pl.ds(vec[k], 1)],
                                 rows_v.at[pl.ds(j * L + k, 1)], sem)
            return _

        lax.fori_loop(0, b_per_w // L, issue, 0, unroll=False)
        pltpu.make_async_copy(tab.at[pl.ds(0, b_per_w)], rows_v, sem).wait()
        pltpu.sync_copy(rows_v, out.at[pl.ds(base, b_per_w)])


@functools.lru_cache(maxsize=None)
def _make_sc_rest():
    info = plsc.get_sparse_core_info()
    b_per_w = B // (info.num_cores * info.num_subcores)
    mesh = plsc.VectorSubcoreMesh(core_axis_name="c", subcore_axis_name="s")
    return pl.kernel(
        functools.partial(_sc_rest_body, b_per_w),
        mesh=mesh,
        out_type=[jax.ShapeDtypeStruct((B, D), jnp.float32)] * 3,
        scratch_types=[
            pltpu.VMEM((b_per_w,), jnp.int32),
            pltpu.VMEM((b_per_w, D), jnp.float32),
            pltpu.SemaphoreType.DMA,
        ],
    )


def _mlp_body(ce, pe, ty, ca, pw, pp,
              pw_w, pw_b, pp_w, pp_b,
              fc1_w, fc1_b, fc2_w, fc2_b, out_w, out_b, o_ref):
    f32 = jnp.float32
    dn = (((1,), (1,)), ((), ()))
    dnt = (((0,), (1,)), ((), ()))

    def bdot(a, b, d=dn):
        return lax.dot_general(a.astype(jnp.bfloat16), b.astype(jnp.bfloat16),
                               d, preferred_element_type=f32)

    power_emb = pw[...] * pw_w[...] + pw_b[...]
    price_emb = pp[...] * pp_w[...] + pp_b[...]
    x = jnp.concatenate(
        [ce[...], pe[...], ty[...], power_emb, ca[...], price_emb], axis=-1)
    h = jnp.maximum(bdot(x, fc1_w[...]) + fc1_b[...], 0.0)
    h = jnp.maximum(bdot(h, fc2_w[...]) + fc2_b[...], 0.0)
    o = jnp.sum(h * out_w[...], axis=1)
    o_ref[...] = o + out_b[0]


def _mlp(ce, pe, ty, ca, pw2, pp2, pw_w, pw_b, pp_w, pp_b,
         fc1_w, fc1_b, fc2_w, fc2_b, out_w, out_b):
    blk = 2048
    grid = (B // blk,)

    def row_spec(d):
        return pl.BlockSpec((blk, d), lambda i: (i, 0))

    def full_spec(shape):
        nd = len(shape)
        return pl.BlockSpec(shape, (lambda i: (0,) * nd))

    in_specs = [
        row_spec(D), row_spec(D), row_spec(D), row_spec(D),
        row_spec(1), row_spec(1),
        full_spec(pw_w.shape), full_spec(pw_b.shape),
        full_spec(pp_w.shape), full_spec(pp_b.shape),
        full_spec(fc1_w.shape), full_spec(fc1_b.shape),
        full_spec(fc2_w.shape), full_spec(fc2_b.shape),
        full_spec(out_w.shape), full_spec(out_b.shape),
    ]
    return pl.pallas_call(
        _mlp_body,
        grid=grid,
        in_specs=in_specs,
        out_specs=pl.BlockSpec((blk,), lambda i: (i,)),
        out_shape=jax.ShapeDtypeStruct((B,), jnp.float32),
    )(ce, pe, ty, ca, pw2, pp2, pw_w, pw_b, pp_w, pp_b,
      fc1_w, fc1_b, fc2_w, fc2_b, out_w, out_b)


def kernel(customer_id, product_id, customer_type, purchasing_power,
           product_category, product_price,
           ce_table, pe_table, type_table, cat_table,
           pw_w, pw_b, pp_w, pp_b,
           fc1_w, fc1_b, fc2_w, fc2_b, out_w, out_b):
    pe, ty, ca = _make_sc_rest()(
        pe_table, type_table, cat_table,
        product_id, customer_type, product_category)
    ce = _make_sc_ce()(ce_table, customer_id)
    return _mlp(ce, pe, ty, ca,
                purchasing_power[:, None], product_price[:, None],
                pw_w.T, pw_b, pp_w.T, pp_b,
                fc1_w, fc1_b, fc2_w, fc2_b, out_w, out_b)


# speedup vs baseline: 1.6080x; 1.0190x over previous
import functools

import jax
import jax.numpy as jnp
from jax import lax
from jax.experimental import pallas as pl
from jax.experimental.pallas import tpu as pltpu
from jax.experimental.pallas import tpu_sc as plsc

B = 16384
D = 64
L = 16


def _worker_id():
    return lax.axis_index("s") * 2 + lax.axis_index("c")


def _sc_ce_body(b_per_w, ce_t, cid, o_ce, idx_v, rows_v, sem):
    base = _worker_id() * b_per_w
    pltpu.sync_copy(cid.at[pl.ds(base, b_per_w)], idx_v)

    def issue(j, _):
        vec = idx_v[pl.ds(j * L, L)]
        for k in range(L):
            pltpu.async_copy(ce_t.at[pl.ds(vec[k], 1)],
                             rows_v.at[pl.ds(j * L + k, 1)], sem)
        return _

    lax.fori_loop(0, b_per_w // L, issue, 0, unroll=False)
    pltpu.make_async_copy(ce_t.at[pl.ds(0, b_per_w)], rows_v, sem).wait()
    pltpu.sync_copy(rows_v, o_ce.at[pl.ds(base, b_per_w)])


@functools.lru_cache(maxsize=None)
def _make_sc_ce():
    info = plsc.get_sparse_core_info()
    b_per_w = B // (info.num_cores * info.num_subcores)
    mesh = plsc.VectorSubcoreMesh(core_axis_name="c", subcore_axis_name="s")
    return pl.kernel(
        functools.partial(_sc_ce_body, b_per_w),
        mesh=mesh,
        out_type=jax.ShapeDtypeStruct((B, D), jnp.float32),
        scratch_types=[
            pltpu.VMEM((b_per_w,), jnp.int32),
            pltpu.VMEM((b_per_w, D), jnp.float32),
            pltpu.SemaphoreType.DMA,
        ],
    )


def _sc_rest_body(b_per_w, pe_t, ty_t, ca_t, pid, tid, gid,
                  o_pe, o_ty, o_ca, idx_v, rows_v, sem):
    base = _worker_id() * b_per_w
    for tab, idx, out in ((pe_t, pid, o_pe), (ty_t, tid, o_ty),
                          (ca_t, gid, o_ca)):
        pltpu.sync_copy(idx.at[pl.ds(base, b_per_w)], idx_v)

        def issue(j, _, tab=tab):
            vec = idx_v[pl.ds(j * L, L)]
            for k in range(L):
                pltpu.async_copy(tab.at[pl.ds(vec[k], 1)],
                                 rows_v.at[pl.ds(j * L + k, 1)], sem)
            return _

        lax.fori_loop(0, b_per_w // L, issue, 0, unroll=False)
        pltpu.make_async_copy(tab.at[pl.ds(0, b_per_w)], rows_v, sem).wait()
        pltpu.sync_copy(rows_v, out.at[pl.ds(base, b_per_w)])


@functools.lru_cache(maxsize=None)
def _make_sc_rest():
    info = plsc.get_sparse_core_info()
    b_per_w = B // (info.num_cores * info.num_subcores)
    mesh = plsc.VectorSubcoreMesh(core_axis_name="c", subcore_axis_name="s")
    return pl.kernel(
        functools.partial(_sc_rest_body, b_per_w),
        mesh=mesh,
        out_type=[jax.ShapeDtypeStruct((B, D), jnp.float32)] * 3,
        scratch_types=[
            pltpu.VMEM((b_per_w,), jnp.int32),
            pltpu.VMEM((b_per_w, D), jnp.float32),
            pltpu.SemaphoreType.DMA,
        ],
    )


def _mlp_body(ce, pe, ty, ca, pw, pp,
              pw_w, pw_b, pp_w, pp_b,
              fc1_w, fc1_b, fc2_w, fc2_b, out_w, out_b, o_ref):
    f32 = jnp.float32
    dn = (((1,), (1,)), ((), ()))
    dnt = (((0,), (1,)), ((), ()))

    def bdot(a, b, d=dn):
        return lax.dot_general(a.astype(jnp.bfloat16), b.astype(jnp.bfloat16),
                               d, preferred_element_type=f32)

    power_emb = pw[...][:, None] * pw_w[...] + pw_b[...]
    price_emb = pp[...][:, None] * pp_w[...] + pp_b[...]
    x = jnp.concatenate(
        [ce[...], pe[...], ty[...], power_emb, ca[...], price_emb], axis=-1)
    h = jnp.maximum(bdot(x, fc1_w[...]) + fc1_b[...], 0.0)
    h = jnp.maximum(bdot(h, fc2_w[...]) + fc2_b[...], 0.0)
    o = jnp.sum(h * out_w[...], axis=1)
    o_ref[...] = o + out_b[0]


def _mlp(ce, pe, ty, ca, pw2, pp2, pw_w, pw_b, pp_w, pp_b,
         fc1_w, fc1_b, fc2_w, fc2_b, out_w, out_b):
    blk = 2048
    grid = (B // blk,)

    def row_spec(d):
        return pl.BlockSpec((blk, d), lambda i: (i, 0))

    def full_spec(shape):
        nd = len(shape)
        return pl.BlockSpec(shape, (lambda i: (0,) * nd))

    vec_spec = pl.BlockSpec((blk,), lambda i: (i,))
    in_specs = [
        row_spec(D), row_spec(D), row_spec(D), row_spec(D),
        vec_spec, vec_spec,
        full_spec(pw_w.shape), full_spec(pw_b.shape),
        full_spec(pp_w.shape), full_spec(pp_b.shape),
        full_spec(fc1_w.shape), full_spec(fc1_b.shape),
        full_spec(fc2_w.shape), full_spec(fc2_b.shape),
        full_spec(out_w.shape), full_spec(out_b.shape),
    ]
    return pl.pallas_call(
        _mlp_body,
        grid=grid,
        in_specs=in_specs,
        out_specs=pl.BlockSpec((blk,), lambda i: (i,)),
        out_shape=jax.ShapeDtypeStruct((B,), jnp.float32),
    )(ce, pe, ty, ca, pw2, pp2, pw_w, pw_b, pp_w, pp_b,
      fc1_w, fc1_b, fc2_w, fc2_b, out_w, out_b)


def kernel(customer_id, product_id, customer_type, purchasing_power,
           product_category, product_price,
           ce_table, pe_table, type_table, cat_table,
           pw_w, pw_b, pp_w, pp_b,
           fc1_w, fc1_b, fc2_w, fc2_b, out_w, out_b):
    pe, ty, ca = _make_sc_rest()(
        pe_table, type_table, cat_table,
        product_id, customer_type, product_category)
    ce = _make_sc_ce()(ce_table, customer_id)
    return _mlp(ce, pe, ty, ca,
                purchasing_power, product_price,
                pw_w.T, pw_b, pp_w.T, pp_b,
                fc1_w, fc1_b, fc2_w, fc2_b, out_w, out_b)
